# baseline (speedup 1.0000x reference)
import jax
import jax.numpy as jnp
from jax.experimental import pallas as pl
from jax.experimental.pallas import tpu as pltpu

N = 10000
B = 64
IN_SIZE = 128
EMB = 128
ROW_TILE = 200
NUM_TILES = (N // 2) // ROW_TILE


def _norm_relu(z):
    r = jnp.maximum(z, 0.0)
    nrm = jnp.sqrt(jnp.sum(r * r, axis=-1, keepdims=True))
    return r / jnp.maximum(nrm, 1e-12)


def _body(a1_ref, a2_ref, nbt1_ref, nbt2_ref, x_ref, w_in_t_ref, b_in_ref,
          w_gcn_ref, w_g_t_ref, b_g_ref, hs1_ref, hs2_ref, hg_ref, h_vmem,
          pooled):
    i = pl.program_id(0)

    @pl.when(i == 0)
    def _init():
        h_vmem[...] = jnp.maximum(
            jnp.dot(x_ref[...], w_in_t_ref[...],
                    preferred_element_type=jnp.float32) + b_in_ref[...],
            0.0)
        pooled[...] = jnp.zeros_like(pooled)

    h = h_vmem[...]
    acc = pooled[...]
    for a_ref, nbt_ref, hs_ref in ((a1_ref, nbt1_ref, hs1_ref),
                                   (a2_ref, nbt2_ref, hs2_ref)):
        z = jnp.dot(a_ref[...], h, preferred_element_type=jnp.float32)
        z = jnp.dot(z, w_gcn_ref[...], preferred_element_type=jnp.float32)
        hs = _norm_relu(z)
        hs_ref[...] = hs
        acc = acc + jax.lax.dot_general(
            nbt_ref[...], hs, (((0,), (0,)), ((), ())),
            preferred_element_type=jnp.float32)
    pooled[...] = acc

    @pl.when(i == NUM_TILES - 1)
    def _finish():
        g = jnp.dot(pooled[...], w_g_t_ref[...],
                    preferred_element_type=jnp.float32) + b_g_ref[...]
        hg_ref[...] = _norm_relu(g)


@jax.jit
def kernel(node_matrix, node_batch, input_node_features, W_in, b_in, W_gcn,
           W_g, b_g):
    nbt = node_batch.T
    b_in2 = b_in.reshape(1, EMB)
    b_g2 = b_g.reshape(1, EMB)
    w_in_t = W_in.T
    w_g_t = W_g.T
    half = NUM_TILES

    grid = (NUM_TILES,)
    hs1, hs2, h_graph = pl.pallas_call(
        _body,
        grid=grid,
        in_specs=[
            pl.BlockSpec((ROW_TILE, N), lambda i: (i, 0)),
            pl.BlockSpec((ROW_TILE, N), lambda i: (i + half, 0)),
            pl.BlockSpec((ROW_TILE, B), lambda i: (i, 0)),
            pl.BlockSpec((ROW_TILE, B), lambda i: (i + half, 0)),
            pl.BlockSpec((N, IN_SIZE), lambda i: (0, 0)),
            pl.BlockSpec((IN_SIZE, EMB), lambda i: (0, 0)),
            pl.BlockSpec((1, EMB), lambda i: (0, 0)),
            pl.BlockSpec((EMB, EMB), lambda i: (0, 0)),
            pl.BlockSpec((EMB, EMB), lambda i: (0, 0)),
            pl.BlockSpec((1, EMB), lambda i: (0, 0)),
        ],
        out_specs=[
            pl.BlockSpec((ROW_TILE, EMB), lambda i: (i, 0)),
            pl.BlockSpec((ROW_TILE, EMB), lambda i: (i, 0)),
            pl.BlockSpec((B, EMB), lambda i: (0, 0)),
        ],
        out_shape=[
            jax.ShapeDtypeStruct((N // 2, EMB), jnp.float32),
            jax.ShapeDtypeStruct((N // 2, EMB), jnp.float32),
            jax.ShapeDtypeStruct((B, EMB), jnp.float32),
        ],
        scratch_shapes=[
            pltpu.VMEM((N, EMB), jnp.float32),
            pltpu.VMEM((B, EMB), jnp.float32),
        ],
    )(node_matrix, node_matrix, nbt, nbt, input_node_features, w_in_t, b_in2,
      W_gcn, w_g_t, b_g2)
    h_struct = jnp.concatenate([hs1, hs2], axis=0)
    return (h_struct, h_graph)


# speedup vs baseline: 1.1851x; 1.1851x over previous
import jax
import jax.numpy as jnp
from jax.experimental import pallas as pl
from jax.experimental.pallas import tpu as pltpu

N = 10000
B = 64
IN_SIZE = 128
EMB = 128
ROW_TILE = 400
NUM_TILES = N // ROW_TILE


def _body(a_ref, nbt_ref, x_ref, w_in_t_ref, b_in_ref, w_gcn_ref, w_g_t_ref,
          b_g_ref, hs_ref, hg_ref, h_vmem, pooled):
    i = pl.program_id(0)

    @pl.when(i == 0)
    def _init():
        h_vmem[...] = jnp.maximum(
            jnp.dot(x_ref[...], w_in_t_ref[...],
                    preferred_element_type=jnp.float32) + b_in_ref[...],
            0.0)
        pooled[...] = jnp.zeros_like(pooled)

    z = jnp.dot(a_ref[...], h_vmem[...], preferred_element_type=jnp.float32)
    z = jnp.dot(z, w_gcn_ref[...], preferred_element_type=jnp.float32)
    r = jnp.maximum(z, 0.0)
    nrm = jnp.sqrt(jnp.sum(r * r, axis=-1, keepdims=True))
    hs = r / jnp.maximum(nrm, 1e-12)
    hs_ref[...] = hs

    pooled[...] += jax.lax.dot_general(
        nbt_ref[...], hs, (((0,), (0,)), ((), ())),
        preferred_element_type=jnp.float32)

    @pl.when(i == NUM_TILES - 1)
    def _finish():
        g = jnp.dot(pooled[...], w_g_t_ref[...],
                    preferred_element_type=jnp.float32) + b_g_ref[...]
        g = jnp.maximum(g, 0.0)
        nrm2 = jnp.sqrt(jnp.sum(g * g, axis=-1, keepdims=True))
        hg_ref[...] = g / jnp.maximum(nrm2, 1e-12)


@jax.jit
def kernel(node_matrix, node_batch, input_node_features, W_in, b_in, W_gcn,
           W_g, b_g):
    nbt = node_batch.T
    b_in2 = b_in.reshape(1, EMB)
    b_g2 = b_g.reshape(1, EMB)
    w_in_t = W_in.T
    w_g_t = W_g.T

    grid = (NUM_TILES,)
    h_struct, h_graph = pl.pallas_call(
        _body,
        grid=grid,
        in_specs=[
            pl.BlockSpec((ROW_TILE, N), lambda i: (i, 0)),
            pl.BlockSpec((ROW_TILE, B), lambda i: (i, 0)),
            pl.BlockSpec((N, IN_SIZE), lambda i: (0, 0)),
            pl.BlockSpec((IN_SIZE, EMB), lambda i: (0, 0)),
            pl.BlockSpec((1, EMB), lambda i: (0, 0)),
            pl.BlockSpec((EMB, EMB), lambda i: (0, 0)),
            pl.BlockSpec((EMB, EMB), lambda i: (0, 0)),
            pl.BlockSpec((1, EMB), lambda i: (0, 0)),
        ],
        out_specs=[
            pl.BlockSpec((ROW_TILE, EMB), lambda i: (i, 0)),
            pl.BlockSpec((B, EMB), lambda i: (0, 0)),
        ],
        out_shape=[
            jax.ShapeDtypeStruct((N, EMB), jnp.float32),
            jax.ShapeDtypeStruct((B, EMB), jnp.float32),
        ],
        scratch_shapes=[
            pltpu.VMEM((N, EMB), jnp.float32),
            pltpu.VMEM((B, EMB), jnp.float32),
        ],
    )(node_matrix, nbt, input_node_features, w_in_t, b_in2, W_gcn, w_g_t,
      b_g2)
    return (h_struct, h_graph)


# speedup vs baseline: 1.2229x; 1.0320x over previous
import jax
import jax.numpy as jnp
from jax.experimental import pallas as pl
from jax.experimental.pallas import tpu as pltpu

N = 10000
B = 64
IN_SIZE = 128
EMB = 128
ROW_TILE = 400
NUM_TILES = N // ROW_TILE


def _body(a_ref, nb_ref, x_ref, w_in_t_ref, b_in_ref, w_gcn_ref, w_g_t_ref,
          b_g_ref, hs_ref, hg_ref, h2_vmem, hs_vmem):
    i = pl.program_id(0)

    @pl.when(i == 0)
    def _init():
        h = jnp.maximum(
            jnp.dot(x_ref[...], w_in_t_ref[...],
                    preferred_element_type=jnp.float32) + b_in_ref[...],
            0.0)
        h2_vmem[...] = jnp.dot(h, w_gcn_ref[...],
                               preferred_element_type=jnp.float32)

    z = jnp.dot(a_ref[...], h2_vmem[...], preferred_element_type=jnp.float32)
    r = jnp.maximum(z, 0.0)
    nrm = jnp.sqrt(jnp.sum(r * r, axis=-1, keepdims=True))
    hs = r / jnp.maximum(nrm, 1e-12)
    hs_ref[...] = hs
    hs_vmem[pl.ds(i * ROW_TILE, ROW_TILE), :] = hs

    @pl.when(i == NUM_TILES - 1)
    def _finish():
        pooled = jnp.dot(nb_ref[...], hs_vmem[...],
                         preferred_element_type=jnp.float32)
        g = jnp.dot(pooled, w_g_t_ref[...],
                    preferred_element_type=jnp.float32) + b_g_ref[...]
        g = jnp.maximum(g, 0.0)
        nrm2 = jnp.sqrt(jnp.sum(g * g, axis=-1, keepdims=True))
        hg_ref[...] = g / jnp.maximum(nrm2, 1e-12)


@jax.jit
def kernel(node_matrix, node_batch, input_node_features, W_in, b_in, W_gcn,
           W_g, b_g):
    b_in2 = b_in.reshape(1, EMB)
    b_g2 = b_g.reshape(1, EMB)
    w_in_t = W_in.T
    w_g_t = W_g.T

    grid = (NUM_TILES,)
    h_struct, h_graph = pl.pallas_call(
        _body,
        grid=grid,
        in_specs=[
            pl.BlockSpec((ROW_TILE, N), lambda i: (i, 0)),
            pl.BlockSpec((B, N), lambda i: (0, 0)),
            pl.BlockSpec((N, IN_SIZE), lambda i: (0, 0)),
            pl.BlockSpec((IN_SIZE, EMB), lambda i: (0, 0)),
            pl.BlockSpec((1, EMB), lambda i: (0, 0)),
            pl.BlockSpec((EMB, EMB), lambda i: (0, 0)),
            pl.BlockSpec((EMB, EMB), lambda i: (0, 0)),
            pl.BlockSpec((1, EMB), lambda i: (0, 0)),
        ],
        out_specs=[
            pl.BlockSpec((ROW_TILE, EMB), lambda i: (i, 0)),
            pl.BlockSpec((B, EMB), lambda i: (0, 0)),
        ],
        out_shape=[
            jax.ShapeDtypeStruct((N, EMB), jnp.float32),
            jax.ShapeDtypeStruct((B, EMB), jnp.float32),
        ],
        scratch_shapes=[
            pltpu.VMEM((N, EMB), jnp.float32),
            pltpu.VMEM((N, EMB), jnp.float32),
        ],
    )(node_matrix, node_batch, input_node_features, w_in_t, b_in2, W_gcn,
      w_g_t, b_g2)
    return (h_struct, h_graph)


# speedup vs baseline: 1.2356x; 1.0104x over previous
import jax
import jax.numpy as jnp
from jax.experimental import pallas as pl
from jax.experimental.pallas import tpu as pltpu

N = 10000
B = 64
IN_SIZE = 128
EMB = 128
ROW_TILE = 400
NUM_TILES = N // ROW_TILE


def _body(a_ref, nb_ref, x_ref, w_in_t_ref, b_in_ref, w_gcn_ref, w_g_t_ref,
          b_g_ref, hs_ref, hg_ref, h2_vmem, hs_vmem):
    i = pl.program_id(0)

    @pl.when(i == 0)
    def _init():
        h2_vmem[...] = jnp.maximum(
            jnp.dot(x_ref[...], w_in_t_ref[...],
                    preferred_element_type=jnp.float32) + b_in_ref[...],
            0.0)

    z = jnp.dot(a_ref[...], h2_vmem[...], preferred_element_type=jnp.float32)
    z = jnp.dot(z, w_gcn_ref[...], preferred_element_type=jnp.float32)
    r = jnp.maximum(z, 0.0)
    nrm = jnp.sqrt(jnp.sum(r * r, axis=-1, keepdims=True))
    hs = r / jnp.maximum(nrm, 1e-12)
    hs_ref[...] = hs
    hs_vmem[pl.ds(i * ROW_TILE, ROW_TILE), :] = hs

    @pl.when(i == NUM_TILES - 1)
    def _finish():
        pooled = jnp.dot(nb_ref[...], hs_vmem[...],
                         preferred_element_type=jnp.float32)
        g = jnp.dot(pooled, w_g_t_ref[...],
                    preferred_element_type=jnp.float32) + b_g_ref[...]
        g = jnp.maximum(g, 0.0)
        nrm2 = jnp.sqrt(jnp.sum(g * g, axis=-1, keepdims=True))
        hg_ref[...] = g / jnp.maximum(nrm2, 1e-12)


@jax.jit
def kernel(node_matrix, node_batch, input_node_features, W_in, b_in, W_gcn,
           W_g, b_g):
    b_in2 = b_in.reshape(1, EMB)
    b_g2 = b_g.reshape(1, EMB)
    w_in_t = W_in.T
    w_g_t = W_g.T

    grid = (NUM_TILES,)
    h_struct, h_graph = pl.pallas_call(
        _body,
        grid=grid,
        in_specs=[
            pl.BlockSpec((ROW_TILE, N), lambda i: (i, 0)),
            pl.BlockSpec((B, N), lambda i: (0, 0)),
            pl.BlockSpec((N, IN_SIZE), lambda i: (0, 0)),
            pl.BlockSpec((IN_SIZE, EMB), lambda i: (0, 0)),
            pl.BlockSpec((1, EMB), lambda i: (0, 0)),
            pl.BlockSpec((EMB, EMB), lambda i: (0, 0)),
            pl.BlockSpec((EMB, EMB), lambda i: (0, 0)),
            pl.BlockSpec((1, EMB), lambda i: (0, 0)),
        ],
        out_specs=[
            pl.BlockSpec((ROW_TILE, EMB), lambda i: (i, 0)),
            pl.BlockSpec((B, EMB), lambda i: (0, 0)),
        ],
        out_shape=[
            jax.ShapeDtypeStruct((N, EMB), jnp.float32),
            jax.ShapeDtypeStruct((B, EMB), jnp.float32),
        ],
        scratch_shapes=[
            pltpu.VMEM((N, EMB), jnp.float32),
            pltpu.VMEM((N, EMB), jnp.float32),
        ],
    )(node_matrix, node_batch, input_node_features, w_in_t, b_in2, W_gcn,
      w_g_t, b_g2)
    return (h_struct, h_graph)


# speedup vs baseline: 1.2644x; 1.0233x over previous
import jax
import jax.numpy as jnp
from jax.experimental import pallas as pl
from jax.experimental.pallas import tpu as pltpu

N = 10000
B = 64
IN_SIZE = 128
EMB = 128
ROW_TILE = 400
NUM_TILES = N // ROW_TILE


def _body(a_ref, nb_ref, x_ref, w_in_t_ref, b_in_ref, w_gcn_ref, w_g_t_ref,
          b_g_ref, hs_ref, hg_ref, h2_vmem, hs_vmem):
    i = pl.program_id(0)

    @pl.when(i == 0)
    def _init():
        h2_vmem[...] = jnp.maximum(
            jax.lax.dot_general(
                x_ref[...], w_in_t_ref[...], (((1,), (1,)), ((), ())),
                preferred_element_type=jnp.float32) + b_in_ref[...],
            0.0)

    z = jnp.dot(a_ref[...], h2_vmem[...], preferred_element_type=jnp.float32)
    z = jnp.dot(z, w_gcn_ref[...], preferred_element_type=jnp.float32)
    r = jnp.maximum(z, 0.0)
    nrm = jnp.sqrt(jnp.sum(r * r, axis=-1, keepdims=True))
    hs = r / jnp.maximum(nrm, 1e-12)
    hs_ref[...] = hs
    hs_vmem[pl.ds(i * ROW_TILE, ROW_TILE), :] = hs

    @pl.when(i == NUM_TILES - 1)
    def _finish():
        pooled = jnp.dot(nb_ref[...], hs_vmem[...],
                         preferred_element_type=jnp.float32)
        g = jax.lax.dot_general(
            pooled, w_g_t_ref[...], (((1,), (1,)), ((), ())),
            preferred_element_type=jnp.float32) + b_g_ref[...]
        g = jnp.maximum(g, 0.0)
        nrm2 = jnp.sqrt(jnp.sum(g * g, axis=-1, keepdims=True))
        hg_ref[...] = g / jnp.maximum(nrm2, 1e-12)


@jax.jit
def kernel(node_matrix, node_batch, input_node_features, W_in, b_in, W_gcn,
           W_g, b_g):
    b_in2 = b_in.reshape(1, EMB)
    b_g2 = b_g.reshape(1, EMB)

    grid = (NUM_TILES,)
    h_struct, h_graph = pl.pallas_call(
        _body,
        grid=grid,
        in_specs=[
            pl.BlockSpec((ROW_TILE, N), lambda i: (i, 0)),
            pl.BlockSpec((B, N), lambda i: (0, 0)),
            pl.BlockSpec((N, IN_SIZE), lambda i: (0, 0)),
            pl.BlockSpec((EMB, IN_SIZE), lambda i: (0, 0)),
            pl.BlockSpec((1, EMB), lambda i: (0, 0)),
            pl.BlockSpec((EMB, EMB), lambda i: (0, 0)),
            pl.BlockSpec((EMB, EMB), lambda i: (0, 0)),
            pl.BlockSpec((1, EMB), lambda i: (0, 0)),
        ],
        out_specs=[
            pl.BlockSpec((ROW_TILE, EMB), lambda i: (i, 0)),
            pl.BlockSpec((B, EMB), lambda i: (0, 0)),
        ],
        out_shape=[
            jax.ShapeDtypeStruct((N, EMB), jnp.float32),
            jax.ShapeDtypeStruct((B, EMB), jnp.float32),
        ],
        scratch_shapes=[
            pltpu.VMEM((N, EMB), jnp.float32),
            pltpu.VMEM((N, EMB), jnp.float32),
        ],
    )(node_matrix, node_batch, input_node_features, W_in, b_in2, W_gcn,
      W_g, b_g2)
    return (h_struct, h_graph)


# speedup vs baseline: 1.2650x; 1.0005x over previous
import jax
import jax.numpy as jnp
from jax.experimental import pallas as pl
from jax.experimental.pallas import tpu as pltpu

N = 10000
B = 64
IN_SIZE = 128
EMB = 128
ROW_TILE = 400
NUM_TILES = N // ROW_TILE


def _body(a_ref, nb_ref, x_ref, w_in_t_ref, b_in_ref, w_gcn_ref, w_g_t_ref,
          b_g_ref, hs_ref, hg_ref, h2_vmem, hs_vmem):
    i = pl.program_id(0)

    @pl.when(i == 0)
    def _init():
        h2_vmem[...] = jnp.maximum(
            jax.lax.dot_general(
                x_ref[...], w_in_t_ref[...], (((1,), (1,)), ((), ())),
                preferred_element_type=jnp.float32) + b_in_ref[...][None, :],
            0.0)

    z = jnp.dot(a_ref[...], h2_vmem[...], preferred_element_type=jnp.float32)
    z = jnp.dot(z, w_gcn_ref[...], preferred_element_type=jnp.float32)
    r = jnp.maximum(z, 0.0)
    nrm = jnp.sqrt(jnp.sum(r * r, axis=-1, keepdims=True))
    hs = r / jnp.maximum(nrm, 1e-12)
    hs_ref[...] = hs
    hs_vmem[pl.ds(i * ROW_TILE, ROW_TILE), :] = hs

    @pl.when(i == NUM_TILES - 1)
    def _finish():
        pooled = jnp.dot(nb_ref[...], hs_vmem[...],
                         preferred_element_type=jnp.float32)
        g = jax.lax.dot_general(
            pooled, w_g_t_ref[...], (((1,), (1,)), ((), ())),
            preferred_element_type=jnp.float32) + b_g_ref[...][None, :]
        g = jnp.maximum(g, 0.0)
        nrm2 = jnp.sqrt(jnp.sum(g * g, axis=-1, keepdims=True))
        hg_ref[...] = g / jnp.maximum(nrm2, 1e-12)


@jax.jit
def kernel(node_matrix, node_batch, input_node_features, W_in, b_in, W_gcn,
           W_g, b_g):
    grid = (NUM_TILES,)
    h_struct, h_graph = pl.pallas_call(
        _body,
        grid=grid,
        in_specs=[
            pl.BlockSpec((ROW_TILE, N), lambda i: (i, 0)),
            pl.BlockSpec((B, N), lambda i: (0, 0)),
            pl.BlockSpec((N, IN_SIZE), lambda i: (0, 0)),
            pl.BlockSpec((EMB, IN_SIZE), lambda i: (0, 0)),
            pl.BlockSpec((EMB,), lambda i: (0,)),
            pl.BlockSpec((EMB, EMB), lambda i: (0, 0)),
            pl.BlockSpec((EMB, EMB), lambda i: (0, 0)),
            pl.BlockSpec((EMB,), lambda i: (0,)),
        ],
        out_specs=[
            pl.BlockSpec((ROW_TILE, EMB), lambda i: (i, 0)),
            pl.BlockSpec((B, EMB), lambda i: (0, 0)),
        ],
        out_shape=[
            jax.ShapeDtypeStruct((N, EMB), jnp.float32),
            jax.ShapeDtypeStruct((B, EMB), jnp.float32),
        ],
        scratch_shapes=[
            pltpu.VMEM((N, EMB), jnp.float32),
            pltpu.VMEM((N, EMB), jnp.float32),
        ],
    )(node_matrix, node_batch, input_node_features, W_in, b_in, W_gcn,
      W_g, b_g)
    return (h_struct, h_graph)


# speedup vs baseline: 1.2680x; 1.0023x over previous
import jax
import jax.numpy as jnp
from jax.experimental import pallas as pl
from jax.experimental.pallas import tpu as pltpu

N = 10000
B = 64
IN_SIZE = 128
EMB = 128
ROW_TILE = 400
NUM_TILES = N // ROW_TILE


POOL_SPLIT = 6400


def _body(a_ref, nb_ref, x_ref, w_in_t_ref, b_in_ref, w_gcn_ref, w_g_t_ref,
          b_g_ref, hs_ref, hg_ref, h2_vmem, hs_vmem, pooled):
    i = pl.program_id(0)

    @pl.when(i == 0)
    def _init():
        h2_vmem[...] = jnp.maximum(
            jax.lax.dot_general(
                x_ref[...], w_in_t_ref[...], (((1,), (1,)), ((), ())),
                preferred_element_type=jnp.float32) + b_in_ref[...][None, :],
            0.0)

    z = jnp.dot(a_ref[...], h2_vmem[...], preferred_element_type=jnp.float32)
    z = jnp.dot(z, w_gcn_ref[...], preferred_element_type=jnp.float32)
    r = jnp.maximum(z, 0.0)
    nrm = jnp.sqrt(jnp.sum(r * r, axis=-1, keepdims=True))
    hs = r / jnp.maximum(nrm, 1e-12)
    hs_ref[...] = hs
    hs_vmem[pl.ds(i * ROW_TILE, ROW_TILE), :] = hs

    @pl.when(i == POOL_SPLIT // ROW_TILE)
    def _pool_head():
        pooled[...] = jnp.dot(nb_ref[:, :POOL_SPLIT], hs_vmem[:POOL_SPLIT, :],
                              preferred_element_type=jnp.float32)

    @pl.when(i == NUM_TILES - 1)
    def _finish():
        p = pooled[...] + jnp.dot(nb_ref[:, POOL_SPLIT:],
                                  hs_vmem[POOL_SPLIT:, :],
                                  preferred_element_type=jnp.float32)
        g = jax.lax.dot_general(
            p, w_g_t_ref[...], (((1,), (1,)), ((), ())),
            preferred_element_type=jnp.float32) + b_g_ref[...][None, :]
        g = jnp.maximum(g, 0.0)
        nrm2 = jnp.sqrt(jnp.sum(g * g, axis=-1, keepdims=True))
        hg_ref[...] = g / jnp.maximum(nrm2, 1e-12)


@jax.jit
def kernel(node_matrix, node_batch, input_node_features, W_in, b_in, W_gcn,
           W_g, b_g):
    grid = (NUM_TILES,)
    h_struct, h_graph = pl.pallas_call(
        _body,
        grid=grid,
        in_specs=[
            pl.BlockSpec((ROW_TILE, N), lambda i: (i, 0)),
            pl.BlockSpec((B, N), lambda i: (0, 0)),
            pl.BlockSpec((N, IN_SIZE), lambda i: (0, 0)),
            pl.BlockSpec((EMB, IN_SIZE), lambda i: (0, 0)),
            pl.BlockSpec((EMB,), lambda i: (0,)),
            pl.BlockSpec((EMB, EMB), lambda i: (0, 0)),
            pl.BlockSpec((EMB, EMB), lambda i: (0, 0)),
            pl.BlockSpec((EMB,), lambda i: (0,)),
        ],
        out_specs=[
            pl.BlockSpec((ROW_TILE, EMB), lambda i: (i, 0)),
            pl.BlockSpec((B, EMB), lambda i: (0, 0)),
        ],
        out_shape=[
            jax.ShapeDtypeStruct((N, EMB), jnp.float32),
            jax.ShapeDtypeStruct((B, EMB), jnp.float32),
        ],
        scratch_shapes=[
            pltpu.VMEM((N, EMB), jnp.float32),
            pltpu.VMEM((N, EMB), jnp.float32),
            pltpu.VMEM((B, EMB), jnp.float32),
        ],
    )(node_matrix, node_batch, input_node_features, W_in, b_in, W_gcn,
      W_g, b_g)
    return (h_struct, h_graph)
